# baseline (speedup 1.0000x reference)
import functools

import jax
import jax.numpy as jnp
from jax import lax
from jax.experimental import pallas as pl
from jax.experimental.pallas import tpu as pltpu
from jax.experimental.pallas import tpu_sc as plsc

N = 10000
E = 320000
G = 128
H1 = 128
H2 = 64
HEADS = 4
R = 8
C = HEADS * H2

NC, NS, L = 2, 16, 16
NW = NC * NS

KSTRIPE = 5120
KPAD = NS * KSTRIPE
CH = 48
EP = 322560
EPW = EP // NW
EPT = EP // NS
NPAD = 10240
TROW = NPAD // NS
PZ = 128
PZA = 32

_mesh = plsc.VectorSubcoreMesh(core_axis_name="c", subcore_axis_name="s")
_sc_params = pltpu.CompilerParams(needs_layout_passes=False)


def _f32(shape):
    return jax.ShapeDtypeStruct(shape, jnp.float32)


def _rot(v, k, lane):
    idx = jnp.bitwise_and(lane + k, L - 1)
    return v.at[idx].get(mode="promise_in_bounds")


def _bcast(v, i, lane):
    idx = jnp.bitwise_and(lane, 0) + i
    return v.at[idx].get(mode="promise_in_bounds")


def _hsum(v, lane):
    for k in (8, 4, 2, 1):
        v = v + _rot(v, k, lane)
    return v


@functools.partial(
    pl.kernel,
    out_type=[_f32((KPAD,)), _f32((KPAD,))],
    mesh=_mesh,
    compiler_params=_sc_params,
    scratch_types=[
        pltpu.VMEM_SHARED((KPAD,), jnp.float32),
        pltpu.VMEM((CH,), jnp.int32),
        pltpu.VMEM((CH,), jnp.int32),
        pltpu.VMEM((CH,), jnp.int32),
        pltpu.VMEM((CH,), jnp.float32),
        pltpu.VMEM((KSTRIPE,), jnp.float32),
        pltpu.SemaphoreType.DMA,
    ],
)
def _sc_count(dst_h, rt_h, zc_h, out0, out1, cnt_sh, dstv, rtv, keyv, onesv,
              zcv, sem):
    c = lax.axis_index("c")
    s = lax.axis_index("s")
    wid = c * NS + s
    for j in range(CH // L):
        onesv[pl.ds(j * L, L)] = jnp.full((L,), 1.0, jnp.float32)
    pltpu.sync_copy(zc_h, zcv)
    pltpu.sync_copy(zcv, cnt_sh.at[pl.ds(s * KSTRIPE, KSTRIPE)])
    plsc.subcore_barrier()

    def body(ch, carry):
        base = wid * EPW + ch * CH
        d1 = pltpu.async_copy(dst_h.at[pl.ds(base, CH)], dstv, sem)
        d2 = pltpu.async_copy(rt_h.at[pl.ds(base, CH)], rtv, sem)
        d1.wait()
        d2.wait()
        for j in range(CH // L):
            sl = pl.ds(j * L, L)
            keyv[sl] = dstv[sl] * R + rtv[sl]
        pltpu.sync_copy(onesv, cnt_sh.at[keyv], add=True)
        return carry

    lax.fori_loop(0, EPW // CH, body, 0)
    plsc.subcore_barrier()

    pltpu.sync_copy(cnt_sh.at[pl.ds(s * KSTRIPE, KSTRIPE)], zcv)

    @pl.when(c == 0)
    def _():
        pltpu.sync_copy(zcv, out0.at[pl.ds(s * KSTRIPE, KSTRIPE)])

    @pl.when(c == 1)
    def _():
        pltpu.sync_copy(zcv, out1.at[pl.ds(s * KSTRIPE, KSTRIPE)])


def _tc_a_body(x_ref, wrel_ref, wroot_ref, b_ref, cnt_ref,
               xw_ref, root_ref, inv_ref):
    r = pl.program_id(0)
    xw_ref[0] = jnp.dot(x_ref[...], wrel_ref[0],
                        preferred_element_type=jnp.float32)

    @pl.when(r == 0)
    def _():
        root_ref[...] = (
            jnp.dot(x_ref[...], wroot_ref[...],
                    preferred_element_type=jnp.float32) + b_ref[...])
        tot = cnt_ref[0] + cnt_ref[1]
        inv_ref[...] = 1.0 / jnp.maximum(tot, 1.0)


def _tc_a(x, W_rel, W_root, b2, cnt2):
    return pl.pallas_call(
        _tc_a_body,
        grid=(R,),
        in_specs=[
            pl.BlockSpec((N, G), lambda r: (0, 0)),
            pl.BlockSpec((1, G, H1), lambda r: (r, 0, 0)),
            pl.BlockSpec((G, H1), lambda r: (0, 0)),
            pl.BlockSpec((1, H1), lambda r: (0, 0)),
            pl.BlockSpec((2, KPAD // 128, 128), lambda r: (0, 0, 0)),
        ],
        out_specs=[
            pl.BlockSpec((1, N, H1), lambda r: (r, 0, 0)),
            pl.BlockSpec((N, H1), lambda r: (0, 0)),
            pl.BlockSpec((KPAD // 128, 128), lambda r: (0, 0)),
        ],
        out_shape=[
            _f32((R, N, H1)),
            _f32((N, H1)),
            _f32((KPAD // 128, 128)),
        ],
    )(x, W_rel, W_root, b2, cnt2)


@functools.partial(
    pl.kernel,
    out_type=[_f32((NPAD, H1)), _f32((NPAD, H1))],
    mesh=_mesh,
    compiler_params=_sc_params,
    scratch_types=[
        pltpu.VMEM_SHARED((NPAD, H1), jnp.float32),
        [pltpu.VMEM((CH,), jnp.int32)] * 2,
        [pltpu.VMEM((CH,), jnp.int32)] * 2,
        [pltpu.VMEM((CH,), jnp.int32)] * 2,
        [pltpu.VMEM((CH,), jnp.int32)] * 2,
        [pltpu.VMEM((CH,), jnp.int32)] * 2,
        [pltpu.VMEM((CH,), jnp.float32)] * 2,
        [pltpu.VMEM((CH, H1), jnp.float32)] * 2,
        [pltpu.VMEM((CH, H1), jnp.float32)] * 2,
        pltpu.VMEM((PZA, H1), jnp.float32),
        [pltpu.SemaphoreType.DMA] * 2,
        [pltpu.SemaphoreType.DMA] * 2,
        pltpu.SemaphoreType.DMA,
    ],
)
def _sc_rgcn(src_h, dst_h, rt_h, xw_h, inv_h, za_h, out0, out1,
             acc_sh, srcv, dstv, rtv, gidxv, wkeyv, wv, rowsv, srow, zav,
             isem, gsem, ssem):
    c = lax.axis_index("c")
    s = lax.axis_index("s")
    wid = c * NS + s
    lane = lax.iota(jnp.int32, L)
    pltpu.sync_copy(za_h.at[pl.ds(0, PZA)], zav)
    for t in range(TROW // PZA):
        pltpu.sync_copy(zav, acc_sh.at[pl.ds(s * TROW + t * PZA, PZA)])
    plsc.subcore_barrier()

    NCH = EPW // CH
    base0 = wid * EPW

    def load_idx(ch, u):
        b = base0 + ch * CH
        pltpu.async_copy(src_h.at[pl.ds(b, CH)], srcv[u], isem[u])
        pltpu.async_copy(dst_h.at[pl.ds(b, CH)], dstv[u], isem[u])
        pltpu.async_copy(rt_h.at[pl.ds(b, CH)], rtv[u], isem[u])

    def wait_idx(u):
        pltpu.make_async_copy(src_h.at[pl.ds(0, CH)], srcv[u],
                              isem[u]).wait()
        pltpu.make_async_copy(dst_h.at[pl.ds(0, CH)], dstv[u],
                              isem[u]).wait()
        pltpu.make_async_copy(rt_h.at[pl.ds(0, CH)], rtv[u], isem[u]).wait()

    def comp_idx(u):
        for j in range(CH // L):
            sl = pl.ds(j * L, L)
            gidxv[u][sl] = rtv[u][sl] * N + srcv[u][sl]
            wkeyv[u][sl] = dstv[u][sl] * R + rtv[u][sl]

    def fire_gath(u):
        pltpu.async_copy(xw_h.at[gidxv[u]], rowsv[u], gsem[u])
        pltpu.async_copy(inv_h.at[wkeyv[u]], wv[u], gsem[u])

    def wait_gath(u):
        pltpu.make_async_copy(xw_h.at[gidxv[u]], rowsv[u], gsem[u]).wait()
        pltpu.make_async_copy(inv_h.at[wkeyv[u]], wv[u], gsem[u]).wait()

    def fire_scat(u):
        pltpu.async_copy(srow[u], acc_sh.at[dstv[u]], ssem, add=True)

    def wait_scat(u):
        pltpu.make_async_copy(srow[u], acc_sh.at[dstv[u]], ssem).wait()

    def scale(u):
        def group(g, cc):
            wg = wv[u][pl.ds(g * L, L)]
            for i in range(L):
                e = g * L + i
                w = _bcast(wg, i, lane)
                for j in range(H1 // L):
                    sl = pl.ds(j * L, L)
                    srow[u][e, sl] = rowsv[u][e, sl] * w
            return cc

        lax.fori_loop(0, CH // L, group, 0)

    load_idx(0, 0)
    wait_idx(0)
    comp_idx(0)
    fire_gath(0)
    load_idx(1, 1)

    def body(cpair, carry):
        i = 2 * cpair
        last = cpair == NCH // 2 - 1

        wait_idx(1)
        comp_idx(1)
        fire_gath(1)
        wait_gath(0)

        @pl.when(cpair > 0)
        def _():
            wait_scat(1)
        scale(0)
        fire_scat(0)

        @pl.when(jnp.logical_not(last))
        def _():
            load_idx(i + 2, 0)

        @pl.when(jnp.logical_not(last))
        def _():
            wait_idx(0)
            comp_idx(0)
            fire_gath(0)
        wait_gath(1)
        wait_scat(0)
        scale(1)
        fire_scat(1)

        @pl.when(jnp.logical_not(last))
        def _():
            load_idx(i + 3, 1)
        return carry

    lax.fori_loop(0, NCH // 2, body, 0)
    wait_scat(1)
    plsc.subcore_barrier()

    def piece(t, carry):
        rowbase = pl.multiple_of(s * TROW + t * PZA, PZA)
        sl = pl.ds(rowbase, PZA)
        pltpu.sync_copy(acc_sh.at[sl], zav)

        @pl.when(c == 0)
        def _():
            pltpu.sync_copy(zav, out0.at[sl])

        @pl.when(c == 1)
        def _():
            pltpu.sync_copy(zav, out1.at[sl])

        return carry

    lax.fori_loop(0, TROW // PZA, piece, 0)


def _tc_b_body(root_ref, a0_ref, a1_ref, wq_ref, bq_ref, wk_ref, bk_ref,
               wv_ref, bv_ref, ws_ref, bs_ref,
               q_ref, k_ref, v_ref, skip_ref):
    h = root_ref[...] + a0_ref[...] + a1_ref[...]
    q_ref[...] = jnp.dot(h, wq_ref[...],
                         preferred_element_type=jnp.float32) + bq_ref[...]
    k_ref[...] = jnp.dot(h, wk_ref[...],
                         preferred_element_type=jnp.float32) + bk_ref[...]
    v_ref[...] = jnp.dot(h, wv_ref[...],
                         preferred_element_type=jnp.float32) + bv_ref[...]
    skip_ref[...] = jnp.dot(h, ws_ref[...],
                            preferred_element_type=jnp.float32) + bs_ref[...]


def _tc_b(root, acc0, acc1, Wq, bq2, Wk, bk2, Wv, bv2, Ws, bs2):
    RB = 2000
    nb = N // RB
    row = pl.BlockSpec((RB, H1), lambda g, i: (i, 0))
    wsp = pl.BlockSpec((H1, 128), lambda g, i: (0, g))
    bsp = pl.BlockSpec((1, 128), lambda g, i: (0, g))
    osp = pl.BlockSpec((RB, 128), lambda g, i: (g * nb + i, 0))
    return pl.pallas_call(
        _tc_b_body,
        grid=(2, nb),
        in_specs=[row, row, row, wsp, bsp, wsp, bsp, wsp, bsp, wsp, bsp],
        out_specs=[osp, osp, osp,
                   pl.BlockSpec((RB, 128), lambda g, i: (i, g))],
        out_shape=[_f32((2 * N, 128)), _f32((2 * N, 128)),
                   _f32((2 * N, 128)), _f32((N, C))],
    )(root, acc0, acc1, Wq, bq2, Wk, bk2, Wv, bv2, Ws, bs2)


@functools.partial(
    pl.kernel,
    out_type=[_f32((NPAD, 128)), _f32((NPAD, 128))],
    mesh=_mesh,
    compiler_params=_sc_params,
    scratch_types=[
        pltpu.VMEM_SHARED((NPAD, 128), jnp.float32),
        pltpu.VMEM_SHARED((NPAD,), jnp.float32),
        pltpu.VMEM_SHARED((NPAD,), jnp.float32),
        [pltpu.VMEM((CH,), jnp.int32)] * 2,
        [pltpu.VMEM((CH,), jnp.int32)] * 2,
        [pltpu.VMEM((CH,), jnp.int32)] * 2,
        [pltpu.VMEM((CH,), jnp.int32)] * 2,
        [pltpu.VMEM((CH, 128), jnp.float32)] * 2,
        [pltpu.VMEM((CH, 128), jnp.float32)] * 2,
        pltpu.VMEM((CH, 128), jnp.float32),
        pltpu.VMEM((CH, 128), jnp.float32),
        [pltpu.VMEM((CH,), jnp.float32)] * 2,
        [pltpu.VMEM((CH,), jnp.float32)] * 2,
        pltpu.VMEM((PZA, H1), jnp.float32),
        pltpu.VMEM((PZA,), jnp.float32),
        pltpu.VMEM((PZA,), jnp.float32),
        [pltpu.SemaphoreType.DMA] * 2,
        [pltpu.SemaphoreType.DMA] * 2,
        pltpu.SemaphoreType.DMA,
        pltpu.SemaphoreType.DMA,
    ],
)
def _sc_attn(src_h, dst_h, qf_h, kf_h, vf_h, za_h, out0, out1,
             att_sh, den0_sh, den1_sh, srcv, dstv, qidxv, sidxv,
             qr, kr, vr, scb, ebuf0, ebuf1, zav, d0v, d1v,
             isem, gsem, vsem, ssem):
    c = lax.axis_index("c")
    s = lax.axis_index("s")
    coff = c * N
    lane = lax.iota(jnp.int32, L)
    pltpu.sync_copy(za_h.at[pl.ds(0, PZA)], zav)
    for t in range(TROW // PZA):
        pltpu.sync_copy(zav, att_sh.at[pl.ds(s * TROW + t * PZA, PZA)])
    for j in range(PZA // L):
        d0v[pl.ds(j * L, L)] = jnp.zeros((L,), jnp.float32)
    for t in range(TROW // PZA):
        sl = pl.ds(s * TROW + t * PZA, PZA)
        pltpu.sync_copy(d0v, den0_sh.at[sl])
        pltpu.sync_copy(d0v, den1_sh.at[sl])
    plsc.subcore_barrier()

    NCH = EPT // CH
    base0 = s * EPT

    def load_idx(ch, u):
        b = base0 + ch * CH
        pltpu.async_copy(src_h.at[pl.ds(b, CH)], srcv[u], isem[u])
        pltpu.async_copy(dst_h.at[pl.ds(b, CH)], dstv[u], isem[u])

    def wait_idx(u):
        pltpu.make_async_copy(src_h.at[pl.ds(0, CH)], srcv[u],
                              isem[u]).wait()
        pltpu.make_async_copy(dst_h.at[pl.ds(0, CH)], dstv[u],
                              isem[u]).wait()

    def comp_idx(u):
        for j in range(CH // L):
            sl = pl.ds(j * L, L)
            qidxv[u][sl] = jnp.minimum(dstv[u][sl] + coff, 2 * N - 1)
            sidxv[u][sl] = jnp.minimum(srcv[u][sl] + coff, 2 * N - 1)

    def fire_qk(u):
        pltpu.async_copy(qf_h.at[qidxv[u]], qr[u], gsem[u])
        pltpu.async_copy(kf_h.at[sidxv[u]], kr[u], gsem[u])

    def wait_qk(u):
        pltpu.make_async_copy(qf_h.at[qidxv[u]], qr[u], gsem[u]).wait()
        pltpu.make_async_copy(kf_h.at[sidxv[u]], kr[u], gsem[u]).wait()

    def fire_v(u):
        pltpu.async_copy(vf_h.at[sidxv[u]], vr, vsem)

    def wait_v(u):
        pltpu.make_async_copy(vf_h.at[sidxv[u]], vr, vsem).wait()

    def fire_scat(u):
        pltpu.async_copy(scb, att_sh.at[dstv[u]], ssem, add=True)
        pltpu.async_copy(ebuf0[u], den0_sh.at[dstv[u]], ssem, add=True)
        pltpu.async_copy(ebuf1[u], den1_sh.at[dstv[u]], ssem, add=True)

    def wait_scat(u):
        pltpu.make_async_copy(scb, att_sh.at[dstv[u]], ssem).wait()
        pltpu.make_async_copy(ebuf0[u], den0_sh.at[dstv[u]], ssem).wait()
        pltpu.make_async_copy(ebuf1[u], den1_sh.at[dstv[u]], ssem).wait()

    def dots(u):
        def group(g, cc):
            eb0 = jnp.zeros((L,), jnp.float32)
            eb1 = jnp.zeros((L,), jnp.float32)
            for i in range(L):
                e = g * L + i
                qv = [qr[u][e, pl.ds(j * L, L)] for j in range(8)]
                kv = [kr[u][e, pl.ds(j * L, L)] for j in range(8)]
                p0 = (qv[0] * kv[0] + qv[1] * kv[1]
                      + qv[2] * kv[2] + qv[3] * kv[3])
                p1 = (qv[4] * kv[4] + qv[5] * kv[5]
                      + qv[6] * kv[6] + qv[7] * kv[7])
                e0 = jnp.exp(_hsum(p0, lane) * 0.125)
                e1 = jnp.exp(_hsum(p1, lane) * 0.125)
                eb0 = jnp.where(lane == i, e0, eb0)
                eb1 = jnp.where(lane == i, e1, eb1)
            ebuf0[u][pl.ds(g * L, L)] = eb0
            ebuf1[u][pl.ds(g * L, L)] = eb1
            return cc

        lax.fori_loop(0, CH // L, group, 0)

    def scale(u):
        def group(g, cc):
            e0g = ebuf0[u][pl.ds(g * L, L)]
            e1g = ebuf1[u][pl.ds(g * L, L)]
            for i in range(L):
                e = g * L + i
                e0 = _bcast(e0g, i, lane)
                e1 = _bcast(e1g, i, lane)
                for j in range(4):
                    sl = pl.ds(j * L, L)
                    scb[e, sl] = vr[e, sl] * e0
                for j in range(4, 8):
                    sl = pl.ds(j * L, L)
                    scb[e, sl] = vr[e, sl] * e1
            return cc

        lax.fori_loop(0, CH // L, group, 0)

    load_idx(0, 0)
    wait_idx(0)
    comp_idx(0)
    fire_qk(0)
    fire_v(0)
    load_idx(1, 1)

    def body(cpair, carry):
        i = 2 * cpair
        last = cpair == NCH // 2 - 1

        wait_idx(1)
        comp_idx(1)
        fire_qk(1)
        wait_qk(0)
        dots(0)

        @pl.when(cpair > 0)
        def _():
            wait_scat(1)
        wait_v(0)
        scale(0)
        fire_v(1)
        fire_scat(0)

        @pl.when(jnp.logical_not(last))
        def _():
            load_idx(i + 2, 0)

        @pl.when(jnp.logical_not(last))
        def _():
            wait_idx(0)
            comp_idx(0)
            fire_qk(0)
        wait_qk(1)
        dots(1)
        wait_scat(0)
        wait_v(1)
        scale(1)

        @pl.when(jnp.logical_not(last))
        def _():
            fire_v(0)
        fire_scat(1)

        @pl.when(jnp.logical_not(last))
        def _():
            load_idx(i + 3, 1)
        return carry

    lax.fori_loop(0, NCH // 2, body, 0)
    wait_scat(1)
    plsc.subcore_barrier()

    def piece(t, carry):
        rowbase = pl.multiple_of(s * TROW + t * PZA, PZA)
        sl = pl.ds(rowbase, PZA)
        pltpu.sync_copy(att_sh.at[sl], zav)
        pltpu.sync_copy(den0_sh.at[sl], d0v)
        pltpu.sync_copy(den1_sh.at[sl], d1v)

        def norm(g, cc):
            rows = g * L + lane
            i0 = 1.0 / jnp.maximum(d0v[pl.ds(g * L, L)], 1e-16)
            i1 = 1.0 / jnp.maximum(d1v[pl.ds(g * L, L)], 1e-16)
            for j in range(H2):
                cj = jnp.full((L,), j, jnp.int32)
                cj2 = jnp.full((L,), j + H2, jnp.int32)
                plsc.store_scatter(
                    zav, [rows, cj],
                    plsc.load_gather(zav, [rows, cj]) * i0)
                plsc.store_scatter(
                    zav, [rows, cj2],
                    plsc.load_gather(zav, [rows, cj2]) * i1)
            return cc

        lax.fori_loop(0, PZA // L, norm, 0)

        @pl.when(c == 0)
        def _():
            pltpu.sync_copy(zav, out0.at[sl])

        @pl.when(c == 1)
        def _():
            pltpu.sync_copy(zav, out1.at[sl])

        return carry

    lax.fori_loop(0, TROW // PZA, piece, 0)


def _tc_c_body(att0_ref, att1_ref, skip_ref, g_ref, b_ref, out_ref):
    out2 = jnp.concatenate([att0_ref[...], att1_ref[...]],
                           axis=1) + skip_ref[...]
    mu = jnp.mean(out2, axis=0, keepdims=True)
    var = jnp.mean(out2 * out2, axis=0, keepdims=True) - mu * mu
    xn = (out2 - mu) * lax.rsqrt(var + 1e-5)
    y = g_ref[...] * xn + b_ref[...]
    out_ref[...] = jnp.where(y > 0, y, 0.01 * y)


def _tc_c(att0, att1, skip, g2, b2):
    asp = pl.BlockSpec((N, 128), lambda i: (0, 0))
    return pl.pallas_call(
        _tc_c_body,
        grid=(1,),
        in_specs=[asp, asp,
                  pl.BlockSpec((N, C), lambda i: (0, 0)),
                  pl.BlockSpec((1, C), lambda i: (0, 0)),
                  pl.BlockSpec((1, C), lambda i: (0, 0))],
        out_specs=pl.BlockSpec((N, C), lambda i: (0, 0)),
        out_shape=_f32((N, C)),
    )(att0, att1, skip, g2, b2)


def kernel(node_features, edge_index, edge_type, W_rel, W_root, b_rgcn,
           Wq, bq, Wk, bk, Wv, bv, Wskip, bskip, gamma, beta):
    src = edge_index[0].astype(jnp.int32)
    dst = edge_index[1].astype(jnp.int32)
    rt = edge_type.astype(jnp.int32)
    pad = EP - E
    srcp = jnp.concatenate([src, jnp.zeros((pad,), jnp.int32)])
    dstp = jnp.concatenate([dst, jnp.full((pad,), N, jnp.int32)])
    rtp = jnp.concatenate([rt, jnp.zeros((pad,), jnp.int32)])

    zc = jnp.zeros((KSTRIPE,), jnp.float32)
    za = jnp.zeros((PZ, H1), jnp.float32)

    cnt0, cnt1 = _sc_count(dstp, rtp, zc)
    cnt2 = jnp.stack([cnt0, cnt1]).reshape(2, KPAD // 128, 128)

    xw, root, inv = _tc_a(node_features, W_rel, W_root,
                          b_rgcn.reshape(1, H1), cnt2)
    xw2 = xw.reshape(R * N, H1)
    invf = inv.reshape(KPAD)

    acc0, acc1 = _sc_rgcn(srcp, dstp, rtp, xw2, invf, za)

    qf, kf, vf, skip = _tc_b(root, acc0, acc1,
                             Wq, bq.reshape(1, C), Wk, bk.reshape(1, C),
                             Wv, bv.reshape(1, C), Wskip, bskip.reshape(1, C))
    att0, att1 = _sc_attn(srcp, dstp, qf, kf, vf, za)

    return _tc_c(att0, att1, skip, gamma.reshape(1, C), beta.reshape(1, C))


# speedup vs baseline: 1.0427x; 1.0427x over previous
import functools

import jax
import jax.numpy as jnp
from jax import lax
from jax.experimental import pallas as pl
from jax.experimental.pallas import tpu as pltpu
from jax.experimental.pallas import tpu_sc as plsc

N = 10000
E = 320000
G = 128
H1 = 128
H2 = 64
HEADS = 4
R = 8
C = HEADS * H2

NC, NS, L = 2, 16, 16
NW = NC * NS

KSTRIPE = 5120
KPAD = NS * KSTRIPE
CH = 48
EP = 322560
EPW = EP // NW
EPT = EP // NS
NPAD = 10240
TROW = NPAD // NS
PZ = 128
PZA = 16

_mesh = plsc.VectorSubcoreMesh(core_axis_name="c", subcore_axis_name="s")
_sc_params = pltpu.CompilerParams(needs_layout_passes=False)


def _f32(shape):
    return jax.ShapeDtypeStruct(shape, jnp.float32)


def _rot(v, k, lane):
    idx = jnp.bitwise_and(lane + k, L - 1)
    return v.at[idx].get(mode="promise_in_bounds")


def _bcast(v, i, lane):
    idx = jnp.bitwise_and(lane, 0) + i
    return v.at[idx].get(mode="promise_in_bounds")


def _hsum(v, lane):
    for k in (8, 4, 2, 1):
        v = v + _rot(v, k, lane)
    return v


@functools.partial(
    pl.kernel,
    out_type=[_f32((KPAD,)), _f32((KPAD,))],
    mesh=_mesh,
    compiler_params=_sc_params,
    scratch_types=[
        pltpu.VMEM_SHARED((KPAD,), jnp.float32),
        pltpu.VMEM((CH,), jnp.int32),
        pltpu.VMEM((CH,), jnp.int32),
        pltpu.VMEM((CH,), jnp.int32),
        pltpu.VMEM((CH,), jnp.float32),
        pltpu.VMEM((KSTRIPE,), jnp.float32),
        pltpu.SemaphoreType.DMA,
    ],
)
def _sc_count(dst_h, rt_h, zc_h, out0, out1, cnt_sh, dstv, rtv, keyv, onesv,
              zcv, sem):
    c = lax.axis_index("c")
    s = lax.axis_index("s")
    wid = c * NS + s
    for j in range(CH // L):
        onesv[pl.ds(j * L, L)] = jnp.full((L,), 1.0, jnp.float32)
    pltpu.sync_copy(zc_h, zcv)
    pltpu.sync_copy(zcv, cnt_sh.at[pl.ds(s * KSTRIPE, KSTRIPE)])
    plsc.subcore_barrier()

    def body(ch, carry):
        base = wid * EPW + ch * CH
        d1 = pltpu.async_copy(dst_h.at[pl.ds(base, CH)], dstv, sem)
        d2 = pltpu.async_copy(rt_h.at[pl.ds(base, CH)], rtv, sem)
        d1.wait()
        d2.wait()
        for j in range(CH // L):
            sl = pl.ds(j * L, L)
            keyv[sl] = dstv[sl] * R + rtv[sl]
        pltpu.sync_copy(onesv, cnt_sh.at[keyv], add=True)
        return carry

    lax.fori_loop(0, EPW // CH, body, 0)
    plsc.subcore_barrier()

    pltpu.sync_copy(cnt_sh.at[pl.ds(s * KSTRIPE, KSTRIPE)], zcv)

    @pl.when(c == 0)
    def _():
        pltpu.sync_copy(zcv, out0.at[pl.ds(s * KSTRIPE, KSTRIPE)])

    @pl.when(c == 1)
    def _():
        pltpu.sync_copy(zcv, out1.at[pl.ds(s * KSTRIPE, KSTRIPE)])


def _tc_a_body(x_ref, wrel_ref, wroot_ref, b_ref, cnt_ref,
               xw_ref, root_ref, inv_ref):
    r = pl.program_id(0)
    xw_ref[0] = jnp.dot(x_ref[...], wrel_ref[0],
                        preferred_element_type=jnp.float32)

    @pl.when(r == 0)
    def _():
        root_ref[...] = (
            jnp.dot(x_ref[...], wroot_ref[...],
                    preferred_element_type=jnp.float32) + b_ref[...])
        tot = cnt_ref[0] + cnt_ref[1]
        inv_ref[...] = 1.0 / jnp.maximum(tot, 1.0)


def _tc_a(x, W_rel, W_root, b2, cnt2):
    return pl.pallas_call(
        _tc_a_body,
        grid=(R,),
        in_specs=[
            pl.BlockSpec((N, G), lambda r: (0, 0)),
            pl.BlockSpec((1, G, H1), lambda r: (r, 0, 0)),
            pl.BlockSpec((G, H1), lambda r: (0, 0)),
            pl.BlockSpec((1, H1), lambda r: (0, 0)),
            pl.BlockSpec((2, KPAD // 128, 128), lambda r: (0, 0, 0)),
        ],
        out_specs=[
            pl.BlockSpec((1, N, H1), lambda r: (r, 0, 0)),
            pl.BlockSpec((N, H1), lambda r: (0, 0)),
            pl.BlockSpec((KPAD // 128, 128), lambda r: (0, 0)),
        ],
        out_shape=[
            _f32((R, N, H1)),
            _f32((N, H1)),
            _f32((KPAD // 128, 128)),
        ],
    )(x, W_rel, W_root, b2, cnt2)


@functools.partial(
    pl.kernel,
    out_type=[_f32((NPAD, H1)), _f32((NPAD, H1))],
    mesh=_mesh,
    compiler_params=_sc_params,
    scratch_types=[
        pltpu.VMEM_SHARED((NPAD, H1), jnp.float32),
        [pltpu.VMEM((CH,), jnp.int32)] * 2,
        [pltpu.VMEM((CH,), jnp.int32)] * 2,
        [pltpu.VMEM((CH,), jnp.int32)] * 2,
        [pltpu.VMEM((CH,), jnp.int32)] * 2,
        [pltpu.VMEM((CH,), jnp.int32)] * 2,
        [pltpu.VMEM((CH,), jnp.float32)] * 2,
        [pltpu.VMEM((CH, H1), jnp.float32)] * 2,
        [pltpu.VMEM((CH, H1), jnp.float32)] * 2,
        pltpu.VMEM((PZA, H1), jnp.float32),
        [pltpu.SemaphoreType.DMA] * 2,
        [pltpu.SemaphoreType.DMA] * 2,
        pltpu.SemaphoreType.DMA,
    ],
)
def _sc_rgcn(src_h, dst_h, rt_h, xw_h, inv_h, za_h, out0, out1,
             acc_sh, srcv, dstv, rtv, gidxv, wkeyv, wv, rowsv, srow, zav,
             isem, gsem, ssem):
    c = lax.axis_index("c")
    s = lax.axis_index("s")
    wid = c * NS + s
    lane = lax.iota(jnp.int32, L)
    pltpu.sync_copy(za_h.at[pl.ds(0, PZA)], zav)
    for t in range(TROW // PZA):
        pltpu.sync_copy(zav, acc_sh.at[pl.ds(s * TROW + t * PZA, PZA)])
    plsc.subcore_barrier()

    NCH = EPW // CH
    base0 = wid * EPW

    def load_idx(ch, u):
        b = base0 + ch * CH
        pltpu.async_copy(src_h.at[pl.ds(b, CH)], srcv[u], isem[u])
        pltpu.async_copy(dst_h.at[pl.ds(b, CH)], dstv[u], isem[u])
        pltpu.async_copy(rt_h.at[pl.ds(b, CH)], rtv[u], isem[u])

    def wait_idx(u):
        pltpu.make_async_copy(src_h.at[pl.ds(0, CH)], srcv[u],
                              isem[u]).wait()
        pltpu.make_async_copy(dst_h.at[pl.ds(0, CH)], dstv[u],
                              isem[u]).wait()
        pltpu.make_async_copy(rt_h.at[pl.ds(0, CH)], rtv[u], isem[u]).wait()

    def comp_idx(u):
        for j in range(CH // L):
            sl = pl.ds(j * L, L)
            gidxv[u][sl] = rtv[u][sl] * N + srcv[u][sl]
            wkeyv[u][sl] = dstv[u][sl] * R + rtv[u][sl]

    def fire_gath(u):
        pltpu.async_copy(xw_h.at[gidxv[u]], rowsv[u], gsem[u])
        pltpu.async_copy(inv_h.at[wkeyv[u]], wv[u], gsem[u])

    def wait_gath(u):
        pltpu.make_async_copy(xw_h.at[gidxv[u]], rowsv[u], gsem[u]).wait()
        pltpu.make_async_copy(inv_h.at[wkeyv[u]], wv[u], gsem[u]).wait()

    def fire_scat(u):
        pltpu.async_copy(srow[u], acc_sh.at[dstv[u]], ssem, add=True)

    def wait_scat(u):
        pltpu.make_async_copy(srow[u], acc_sh.at[dstv[u]], ssem).wait()

    def scale(u):
        def group(g, cc):
            wg = wv[u][pl.ds(g * L, L)]
            for i in range(L):
                e = g * L + i
                w = _bcast(wg, i, lane)
                for j in range(H1 // L):
                    sl = pl.ds(j * L, L)
                    srow[u][e, sl] = rowsv[u][e, sl] * w
            return cc

        lax.fori_loop(0, CH // L, group, 0)

    load_idx(0, 0)
    wait_idx(0)
    comp_idx(0)
    fire_gath(0)
    load_idx(1, 1)

    def body(cpair, carry):
        i = 2 * cpair
        last = cpair == NCH // 2 - 1

        wait_idx(1)
        comp_idx(1)
        fire_gath(1)
        wait_gath(0)

        @pl.when(cpair > 0)
        def _():
            wait_scat(1)
        scale(0)
        fire_scat(0)

        @pl.when(jnp.logical_not(last))
        def _():
            load_idx(i + 2, 0)

        @pl.when(jnp.logical_not(last))
        def _():
            wait_idx(0)
            comp_idx(0)
            fire_gath(0)
        wait_gath(1)
        wait_scat(0)
        scale(1)
        fire_scat(1)

        @pl.when(jnp.logical_not(last))
        def _():
            load_idx(i + 3, 1)
        return carry

    lax.fori_loop(0, NCH // 2, body, 0)
    wait_scat(1)
    plsc.subcore_barrier()

    def piece(t, carry):
        rowbase = pl.multiple_of(s * TROW + t * PZA, PZA)
        sl = pl.ds(rowbase, PZA)
        pltpu.sync_copy(acc_sh.at[sl], zav)

        @pl.when(c == 0)
        def _():
            pltpu.sync_copy(zav, out0.at[sl])

        @pl.when(c == 1)
        def _():
            pltpu.sync_copy(zav, out1.at[sl])

        return carry

    lax.fori_loop(0, TROW // PZA, piece, 0)


def _tc_b_body(root_ref, a0_ref, a1_ref, wq_ref, bq_ref, wk_ref, bk_ref,
               wv_ref, bv_ref, ws_ref, bs_ref,
               q_ref, k_ref, v_ref, skip_ref):
    h = root_ref[...] + a0_ref[...] + a1_ref[...]
    q_ref[...] = jnp.dot(h, wq_ref[...],
                         preferred_element_type=jnp.float32) + bq_ref[...]
    k_ref[...] = jnp.dot(h, wk_ref[...],
                         preferred_element_type=jnp.float32) + bk_ref[...]
    v_ref[...] = jnp.dot(h, wv_ref[...],
                         preferred_element_type=jnp.float32) + bv_ref[...]
    skip_ref[...] = jnp.dot(h, ws_ref[...],
                            preferred_element_type=jnp.float32) + bs_ref[...]


def _tc_b(root, acc0, acc1, Wq, bq2, Wk, bk2, Wv, bv2, Ws, bs2):
    RB = 2000
    nb = N // RB
    row = pl.BlockSpec((RB, H1), lambda g, i: (i, 0))
    wsp = pl.BlockSpec((H1, 128), lambda g, i: (0, g))
    bsp = pl.BlockSpec((1, 128), lambda g, i: (0, g))
    osp = pl.BlockSpec((RB, 128), lambda g, i: (g * nb + i, 0))
    return pl.pallas_call(
        _tc_b_body,
        grid=(2, nb),
        in_specs=[row, row, row, wsp, bsp, wsp, bsp, wsp, bsp, wsp, bsp],
        out_specs=[osp, osp, osp,
                   pl.BlockSpec((RB, 128), lambda g, i: (i, g))],
        out_shape=[_f32((2 * N, 128)), _f32((2 * N, 128)),
                   _f32((2 * N, 128)), _f32((N, C))],
    )(root, acc0, acc1, Wq, bq2, Wk, bk2, Wv, bv2, Ws, bs2)


@functools.partial(
    pl.kernel,
    out_type=[_f32((NPAD, 128)), _f32((NPAD, 128))],
    mesh=_mesh,
    compiler_params=_sc_params,
    scratch_types=[
        pltpu.VMEM_SHARED((NPAD, 128), jnp.float32),
        pltpu.VMEM_SHARED((NPAD,), jnp.float32),
        pltpu.VMEM_SHARED((NPAD,), jnp.float32),
        [pltpu.VMEM((CH,), jnp.int32)] * 2,
        [pltpu.VMEM((CH,), jnp.int32)] * 2,
        [pltpu.VMEM((CH,), jnp.int32)] * 2,
        [pltpu.VMEM((CH,), jnp.int32)] * 2,
        [pltpu.VMEM((CH, 128), jnp.float32)] * 2,
        [pltpu.VMEM((CH, 128), jnp.float32)] * 2,
        pltpu.VMEM((CH, 128), jnp.float32),
        pltpu.VMEM((CH, 128), jnp.float32),
        [pltpu.VMEM((CH,), jnp.float32)] * 2,
        [pltpu.VMEM((CH,), jnp.float32)] * 2,
        pltpu.VMEM((PZA, H1), jnp.float32),
        pltpu.VMEM((PZA,), jnp.float32),
        pltpu.VMEM((PZA,), jnp.float32),
        [pltpu.SemaphoreType.DMA] * 2,
        [pltpu.SemaphoreType.DMA] * 2,
        pltpu.SemaphoreType.DMA,
        pltpu.SemaphoreType.DMA,
    ],
)
def _sc_attn(src_h, dst_h, qf_h, kf_h, vf_h, za_h, out0, out1,
             att_sh, den0_sh, den1_sh, srcv, dstv, qidxv, sidxv,
             qr, kr, vr, scb, ebuf0, ebuf1, zav, d0v, d1v,
             isem, gsem, vsem, ssem):
    c = lax.axis_index("c")
    s = lax.axis_index("s")
    coff = c * N
    lane = lax.iota(jnp.int32, L)
    pltpu.sync_copy(za_h.at[pl.ds(0, PZA)], zav)
    for t in range(TROW // PZA):
        pltpu.sync_copy(zav, att_sh.at[pl.ds(s * TROW + t * PZA, PZA)])
    for j in range(PZA // L):
        d0v[pl.ds(j * L, L)] = jnp.zeros((L,), jnp.float32)
    for t in range(TROW // PZA):
        sl = pl.ds(s * TROW + t * PZA, PZA)
        pltpu.sync_copy(d0v, den0_sh.at[sl])
        pltpu.sync_copy(d0v, den1_sh.at[sl])
    plsc.subcore_barrier()

    NCH = EPT // CH
    base0 = s * EPT

    def load_idx(ch, u):
        b = base0 + ch * CH
        pltpu.async_copy(src_h.at[pl.ds(b, CH)], srcv[u], isem[u])
        pltpu.async_copy(dst_h.at[pl.ds(b, CH)], dstv[u], isem[u])

    def wait_idx(u):
        pltpu.make_async_copy(src_h.at[pl.ds(0, CH)], srcv[u],
                              isem[u]).wait()
        pltpu.make_async_copy(dst_h.at[pl.ds(0, CH)], dstv[u],
                              isem[u]).wait()

    def comp_idx(u):
        for j in range(CH // L):
            sl = pl.ds(j * L, L)
            qidxv[u][sl] = jnp.minimum(dstv[u][sl] + coff, 2 * N - 1)
            sidxv[u][sl] = jnp.minimum(srcv[u][sl] + coff, 2 * N - 1)

    def fire_qk(u):
        pltpu.async_copy(qf_h.at[qidxv[u]], qr[u], gsem[u])
        pltpu.async_copy(kf_h.at[sidxv[u]], kr[u], gsem[u])

    def wait_qk(u):
        pltpu.make_async_copy(qf_h.at[qidxv[u]], qr[u], gsem[u]).wait()
        pltpu.make_async_copy(kf_h.at[sidxv[u]], kr[u], gsem[u]).wait()

    def fire_v(u):
        pltpu.async_copy(vf_h.at[sidxv[u]], vr, vsem)

    def wait_v(u):
        pltpu.make_async_copy(vf_h.at[sidxv[u]], vr, vsem).wait()

    def fire_scat(u):
        pltpu.async_copy(scb, att_sh.at[dstv[u]], ssem, add=True)
        pltpu.async_copy(ebuf0[u], den0_sh.at[dstv[u]], ssem, add=True)
        pltpu.async_copy(ebuf1[u], den1_sh.at[dstv[u]], ssem, add=True)

    def wait_scat(u):
        pltpu.make_async_copy(scb, att_sh.at[dstv[u]], ssem).wait()
        pltpu.make_async_copy(ebuf0[u], den0_sh.at[dstv[u]], ssem).wait()
        pltpu.make_async_copy(ebuf1[u], den1_sh.at[dstv[u]], ssem).wait()

    def dots(u):
        def group(g, cc):
            eb0 = jnp.zeros((L,), jnp.float32)
            eb1 = jnp.zeros((L,), jnp.float32)
            for i in range(L):
                e = g * L + i
                qv = [qr[u][e, pl.ds(j * L, L)] for j in range(8)]
                kv = [kr[u][e, pl.ds(j * L, L)] for j in range(8)]
                p0 = (qv[0] * kv[0] + qv[1] * kv[1]
                      + qv[2] * kv[2] + qv[3] * kv[3])
                p1 = (qv[4] * kv[4] + qv[5] * kv[5]
                      + qv[6] * kv[6] + qv[7] * kv[7])
                e0 = jnp.exp(_hsum(p0, lane) * 0.125)
                e1 = jnp.exp(_hsum(p1, lane) * 0.125)
                eb0 = jnp.where(lane == i, e0, eb0)
                eb1 = jnp.where(lane == i, e1, eb1)
            ebuf0[u][pl.ds(g * L, L)] = eb0
            ebuf1[u][pl.ds(g * L, L)] = eb1
            return cc

        lax.fori_loop(0, CH // L, group, 0)

    def scale(u):
        def group(g, cc):
            e0g = ebuf0[u][pl.ds(g * L, L)]
            e1g = ebuf1[u][pl.ds(g * L, L)]
            for i in range(L):
                e = g * L + i
                e0 = _bcast(e0g, i, lane)
                e1 = _bcast(e1g, i, lane)
                for j in range(4):
                    sl = pl.ds(j * L, L)
                    scb[e, sl] = vr[e, sl] * e0
                for j in range(4, 8):
                    sl = pl.ds(j * L, L)
                    scb[e, sl] = vr[e, sl] * e1
            return cc

        lax.fori_loop(0, CH // L, group, 0)

    load_idx(0, 0)
    wait_idx(0)
    comp_idx(0)
    fire_qk(0)
    fire_v(0)
    load_idx(1, 1)

    def body(cpair, carry):
        i = 2 * cpair
        last = cpair == NCH // 2 - 1

        wait_idx(1)
        comp_idx(1)
        fire_qk(1)
        wait_qk(0)
        dots(0)

        @pl.when(cpair > 0)
        def _():
            wait_scat(1)
        wait_v(0)
        scale(0)
        fire_v(1)
        fire_scat(0)

        @pl.when(jnp.logical_not(last))
        def _():
            load_idx(i + 2, 0)

        @pl.when(jnp.logical_not(last))
        def _():
            wait_idx(0)
            comp_idx(0)
            fire_qk(0)
        wait_qk(1)
        dots(1)
        wait_scat(0)
        wait_v(1)
        scale(1)

        @pl.when(jnp.logical_not(last))
        def _():
            fire_v(0)
        fire_scat(1)

        @pl.when(jnp.logical_not(last))
        def _():
            load_idx(i + 3, 1)
        return carry

    lax.fori_loop(0, NCH // 2, body, 0)
    wait_scat(1)
    plsc.subcore_barrier()

    def piece(t, carry):
        rowbase = pl.multiple_of(s * TROW + t * PZA, PZA)
        sl = pl.ds(rowbase, PZA)
        pltpu.sync_copy(att_sh.at[sl], zav)
        pltpu.sync_copy(den0_sh.at[sl], d0v)
        pltpu.sync_copy(den1_sh.at[sl], d1v)

        def norm(g, cc):
            rows = g * L + lane
            i0 = 1.0 / jnp.maximum(d0v[pl.ds(g * L, L)], 1e-16)
            i1 = 1.0 / jnp.maximum(d1v[pl.ds(g * L, L)], 1e-16)
            for j in range(H2):
                cj = jnp.full((L,), j, jnp.int32)
                cj2 = jnp.full((L,), j + H2, jnp.int32)
                plsc.store_scatter(
                    zav, [rows, cj],
                    plsc.load_gather(zav, [rows, cj]) * i0)
                plsc.store_scatter(
                    zav, [rows, cj2],
                    plsc.load_gather(zav, [rows, cj2]) * i1)
            return cc

        lax.fori_loop(0, PZA // L, norm, 0)

        @pl.when(c == 0)
        def _():
            pltpu.sync_copy(zav, out0.at[sl])

        @pl.when(c == 1)
        def _():
            pltpu.sync_copy(zav, out1.at[sl])

        return carry

    lax.fori_loop(0, TROW // PZA, piece, 0)


def _tc_c_body(att0_ref, att1_ref, skip_ref, g_ref, b_ref, out_ref):
    out2 = jnp.concatenate([att0_ref[...], att1_ref[...]],
                           axis=1) + skip_ref[...]
    mu = jnp.mean(out2, axis=0, keepdims=True)
    var = jnp.mean(out2 * out2, axis=0, keepdims=True) - mu * mu
    xn = (out2 - mu) * lax.rsqrt(var + 1e-5)
    y = g_ref[...] * xn + b_ref[...]
    out_ref[...] = jnp.where(y > 0, y, 0.01 * y)


def _tc_c(att0, att1, skip, g2, b2):
    asp = pl.BlockSpec((N, 128), lambda i: (0, 0))
    return pl.pallas_call(
        _tc_c_body,
        grid=(1,),
        in_specs=[asp, asp,
                  pl.BlockSpec((N, C), lambda i: (0, 0)),
                  pl.BlockSpec((1, C), lambda i: (0, 0)),
                  pl.BlockSpec((1, C), lambda i: (0, 0))],
        out_specs=pl.BlockSpec((N, C), lambda i: (0, 0)),
        out_shape=_f32((N, C)),
    )(att0, att1, skip, g2, b2)


def kernel(node_features, edge_index, edge_type, W_rel, W_root, b_rgcn,
           Wq, bq, Wk, bk, Wv, bv, Wskip, bskip, gamma, beta):
    src = edge_index[0].astype(jnp.int32)
    dst = edge_index[1].astype(jnp.int32)
    rt = edge_type.astype(jnp.int32)
    pad = EP - E
    srcp = jnp.concatenate([src, jnp.zeros((pad,), jnp.int32)])
    dstp = jnp.concatenate([dst, jnp.full((pad,), N, jnp.int32)])
    rtp = jnp.concatenate([rt, jnp.zeros((pad,), jnp.int32)])

    zc = jnp.zeros((KSTRIPE,), jnp.float32)
    za = jnp.zeros((PZ, H1), jnp.float32)

    cnt0, cnt1 = _sc_count(dstp, rtp, zc)
    cnt2 = jnp.stack([cnt0, cnt1]).reshape(2, KPAD // 128, 128)

    xw, root, inv = _tc_a(node_features, W_rel, W_root,
                          b_rgcn.reshape(1, H1), cnt2)
    xw2 = xw.reshape(R * N, H1)
    invf = inv.reshape(KPAD)

    acc0, acc1 = _sc_rgcn(srcp, dstp, rtp, xw2, invf, za)

    qf, kf, vf, skip = _tc_b(root, acc0, acc1,
                             Wq, bq.reshape(1, C), Wk, bk.reshape(1, C),
                             Wv, bv.reshape(1, C), Wskip, bskip.reshape(1, C))
    att0, att1 = _sc_attn(srcp, dstp, qf, kf, vf, za)

    return _tc_c(att0, att1, skip, gamma.reshape(1, C), beta.reshape(1, C))


# speedup vs baseline: 1.0757x; 1.0316x over previous
import functools

import jax
import jax.numpy as jnp
from jax import lax
from jax.experimental import pallas as pl
from jax.experimental.pallas import tpu as pltpu
from jax.experimental.pallas import tpu_sc as plsc

N = 10000
E = 320000
G = 128
H1 = 128
H2 = 64
HEADS = 4
R = 8
C = HEADS * H2

NC, NS, L = 2, 16, 16
NW = NC * NS

KSTRIPE = 5120
KPAD = NS * KSTRIPE
CH = 48
EP = 322560
EPW = EP // NW
EPT = EP // NS
NPAD = 10240
TROW = NPAD // NS
PZ = 128
PZA = 16

_mesh = plsc.VectorSubcoreMesh(core_axis_name="c", subcore_axis_name="s")
_sc_params = pltpu.CompilerParams(needs_layout_passes=False)


def _f32(shape):
    return jax.ShapeDtypeStruct(shape, jnp.float32)


def _rot(v, k, lane):
    idx = jnp.bitwise_and(lane + k, L - 1)
    return v.at[idx].get(mode="promise_in_bounds")


def _bcast(v, i, lane):
    idx = jnp.bitwise_and(lane, 0) + i
    return v.at[idx].get(mode="promise_in_bounds")


def _hsum(v, lane):
    for k in (8, 4, 2, 1):
        v = v + _rot(v, k, lane)
    return v


@functools.partial(
    pl.kernel,
    out_type=[_f32((KPAD,)), _f32((KPAD,))],
    mesh=_mesh,
    compiler_params=_sc_params,
    scratch_types=[
        pltpu.VMEM_SHARED((KPAD,), jnp.float32),
        pltpu.VMEM((CH,), jnp.int32),
        pltpu.VMEM((CH,), jnp.int32),
        pltpu.VMEM((CH,), jnp.int32),
        pltpu.VMEM((CH,), jnp.float32),
        pltpu.VMEM((KSTRIPE,), jnp.float32),
        pltpu.SemaphoreType.DMA,
    ],
)
def _sc_count(dst_h, rt_h, zc_h, out0, out1, cnt_sh, dstv, rtv, keyv, onesv,
              zcv, sem):
    c = lax.axis_index("c")
    s = lax.axis_index("s")
    wid = c * NS + s
    for j in range(CH // L):
        onesv[pl.ds(j * L, L)] = jnp.full((L,), 1.0, jnp.float32)
    pltpu.sync_copy(zc_h, zcv)
    pltpu.sync_copy(zcv, cnt_sh.at[pl.ds(s * KSTRIPE, KSTRIPE)])
    plsc.subcore_barrier()

    def body(ch, carry):
        base = wid * EPW + ch * CH
        d1 = pltpu.async_copy(dst_h.at[pl.ds(base, CH)], dstv, sem)
        d2 = pltpu.async_copy(rt_h.at[pl.ds(base, CH)], rtv, sem)
        d1.wait()
        d2.wait()
        for j in range(CH // L):
            sl = pl.ds(j * L, L)
            keyv[sl] = dstv[sl] * R + rtv[sl]
        pltpu.sync_copy(onesv, cnt_sh.at[keyv], add=True)
        return carry

    lax.fori_loop(0, EPW // CH, body, 0)
    plsc.subcore_barrier()

    pltpu.sync_copy(cnt_sh.at[pl.ds(s * KSTRIPE, KSTRIPE)], zcv)

    @pl.when(c == 0)
    def _():
        pltpu.sync_copy(zcv, out0.at[pl.ds(s * KSTRIPE, KSTRIPE)])

    @pl.when(c == 1)
    def _():
        pltpu.sync_copy(zcv, out1.at[pl.ds(s * KSTRIPE, KSTRIPE)])


def _tc_a1_body(x_ref, wrel_ref, wroot_ref, b_ref, xw_ref, root_ref):
    r = pl.program_id(0)
    xw_ref[0] = jnp.dot(x_ref[...], wrel_ref[0],
                        preferred_element_type=jnp.float32)

    @pl.when(r == 0)
    def _():
        root_ref[...] = (
            jnp.dot(x_ref[...], wroot_ref[...],
                    preferred_element_type=jnp.float32) + b_ref[...])


def _tc_a1(x, W_rel, W_root, b2):
    return pl.pallas_call(
        _tc_a1_body,
        grid=(R,),
        in_specs=[
            pl.BlockSpec((N, G), lambda r: (0, 0)),
            pl.BlockSpec((1, G, H1), lambda r: (r, 0, 0)),
            pl.BlockSpec((G, H1), lambda r: (0, 0)),
            pl.BlockSpec((1, H1), lambda r: (0, 0)),
        ],
        out_specs=[
            pl.BlockSpec((1, N, H1), lambda r: (r, 0, 0)),
            pl.BlockSpec((N, H1), lambda r: (0, 0)),
        ],
        out_shape=[
            _f32((R, N, H1)),
            _f32((N, H1)),
        ],
    )(x, W_rel, W_root, b2)


def _tc_a2_body(cnt_ref, inv_ref):
    tot = cnt_ref[0] + cnt_ref[1]
    inv_ref[...] = 1.0 / jnp.maximum(tot, 1.0)


def _tc_a2(cnt2):
    return pl.pallas_call(
        _tc_a2_body,
        grid=(1,),
        in_specs=[pl.BlockSpec((2, KPAD // 128, 128), lambda i: (0, 0, 0))],
        out_specs=pl.BlockSpec((KPAD // 128, 128), lambda i: (0, 0)),
        out_shape=_f32((KPAD // 128, 128)),
    )(cnt2)


@functools.partial(
    pl.kernel,
    out_type=[_f32((NPAD, H1)), _f32((NPAD, H1))],
    mesh=_mesh,
    compiler_params=_sc_params,
    scratch_types=[
        pltpu.VMEM_SHARED((NPAD, H1), jnp.float32),
        [pltpu.VMEM((CH,), jnp.int32)] * 2,
        [pltpu.VMEM((CH,), jnp.int32)] * 2,
        [pltpu.VMEM((CH,), jnp.int32)] * 2,
        [pltpu.VMEM((CH,), jnp.int32)] * 2,
        [pltpu.VMEM((CH,), jnp.int32)] * 2,
        [pltpu.VMEM((CH,), jnp.float32)] * 2,
        [pltpu.VMEM((CH, H1), jnp.float32)] * 2,
        [pltpu.VMEM((CH, H1), jnp.float32)] * 2,
        pltpu.VMEM((PZA, H1), jnp.float32),
        [pltpu.SemaphoreType.DMA] * 2,
        [pltpu.SemaphoreType.DMA] * 2,
        pltpu.SemaphoreType.DMA,
    ],
)
def _sc_rgcn(src_h, dst_h, rt_h, xw_h, inv_h, za_h, out0, out1,
             acc_sh, srcv, dstv, rtv, gidxv, wkeyv, wv, rowsv, srow, zav,
             isem, gsem, ssem):
    c = lax.axis_index("c")
    s = lax.axis_index("s")
    wid = c * NS + s
    lane = lax.iota(jnp.int32, L)
    pltpu.sync_copy(za_h.at[pl.ds(0, PZA)], zav)
    for t in range(TROW // PZA):
        pltpu.sync_copy(zav, acc_sh.at[pl.ds(s * TROW + t * PZA, PZA)])
    plsc.subcore_barrier()

    NCH = EPW // CH
    base0 = wid * EPW

    def load_idx(ch, u):
        b = base0 + ch * CH
        pltpu.async_copy(src_h.at[pl.ds(b, CH)], srcv[u], isem[u])
        pltpu.async_copy(dst_h.at[pl.ds(b, CH)], dstv[u], isem[u])
        pltpu.async_copy(rt_h.at[pl.ds(b, CH)], rtv[u], isem[u])

    def wait_idx(u):
        pltpu.make_async_copy(src_h.at[pl.ds(0, CH)], srcv[u],
                              isem[u]).wait()
        pltpu.make_async_copy(dst_h.at[pl.ds(0, CH)], dstv[u],
                              isem[u]).wait()
        pltpu.make_async_copy(rt_h.at[pl.ds(0, CH)], rtv[u], isem[u]).wait()

    def comp_idx(u):
        for j in range(CH // L):
            sl = pl.ds(j * L, L)
            gidxv[u][sl] = rtv[u][sl] * N + srcv[u][sl]
            wkeyv[u][sl] = dstv[u][sl] * R + rtv[u][sl]

    def fire_gath(u):
        pltpu.async_copy(xw_h.at[gidxv[u]], rowsv[u], gsem[u])
        pltpu.async_copy(inv_h.at[wkeyv[u]], wv[u], gsem[u])

    def wait_gath(u):
        pltpu.make_async_copy(xw_h.at[gidxv[u]], rowsv[u], gsem[u]).wait()
        pltpu.make_async_copy(inv_h.at[wkeyv[u]], wv[u], gsem[u]).wait()

    def fire_scat(u):
        pltpu.async_copy(srow[u], acc_sh.at[dstv[u]], ssem, add=True)

    def wait_scat(u):
        pltpu.make_async_copy(srow[u], acc_sh.at[dstv[u]], ssem).wait()

    def scale(u):
        def group(g, cc):
            wg = wv[u][pl.ds(g * L, L)]
            for i in range(L):
                e = g * L + i
                w = _bcast(wg, i, lane)
                for j in range(H1 // L):
                    sl = pl.ds(j * L, L)
                    srow[u][e, sl] = rowsv[u][e, sl] * w
            return cc

        lax.fori_loop(0, CH // L, group, 0)

    load_idx(0, 0)
    wait_idx(0)
    comp_idx(0)
    fire_gath(0)
    load_idx(1, 1)

    def body(cpair, carry):
        i = 2 * cpair
        last = cpair == NCH // 2 - 1

        wait_idx(1)
        comp_idx(1)
        fire_gath(1)
        wait_gath(0)

        @pl.when(cpair > 0)
        def _():
            wait_scat(1)
        scale(0)
        fire_scat(0)

        @pl.when(jnp.logical_not(last))
        def _():
            load_idx(i + 2, 0)

        @pl.when(jnp.logical_not(last))
        def _():
            wait_idx(0)
            comp_idx(0)
            fire_gath(0)
        wait_gath(1)
        wait_scat(0)
        scale(1)
        fire_scat(1)

        @pl.when(jnp.logical_not(last))
        def _():
            load_idx(i + 3, 1)
        return carry

    lax.fori_loop(0, NCH // 2, body, 0)
    wait_scat(1)
    plsc.subcore_barrier()

    def piece(t, carry):
        rowbase = pl.multiple_of(s * TROW + t * PZA, PZA)
        sl = pl.ds(rowbase, PZA)
        pltpu.sync_copy(acc_sh.at[sl], zav)

        @pl.when(c == 0)
        def _():
            pltpu.sync_copy(zav, out0.at[sl])

        @pl.when(c == 1)
        def _():
            pltpu.sync_copy(zav, out1.at[sl])

        return carry

    lax.fori_loop(0, TROW // PZA, piece, 0)


def _tc_b_body(root_ref, a0_ref, a1_ref, wq_ref, bq_ref, wk_ref, bk_ref,
               wv_ref, bv_ref, ws_ref, bs_ref,
               q_ref, k_ref, v_ref, skip_ref):
    h = root_ref[...] + a0_ref[...] + a1_ref[...]
    q_ref[...] = jnp.dot(h, wq_ref[...],
                         preferred_element_type=jnp.float32) + bq_ref[...]
    k_ref[...] = jnp.dot(h, wk_ref[...],
                         preferred_element_type=jnp.float32) + bk_ref[...]
    v_ref[...] = jnp.dot(h, wv_ref[...],
                         preferred_element_type=jnp.float32) + bv_ref[...]
    skip_ref[...] = jnp.dot(h, ws_ref[...],
                            preferred_element_type=jnp.float32) + bs_ref[...]


def _tc_b(root, acc0, acc1, Wq, bq2, Wk, bk2, Wv, bv2, Ws, bs2):
    RB = 2000
    nb = N // RB
    row = pl.BlockSpec((RB, H1), lambda g, i: (i, 0))
    wsp = pl.BlockSpec((H1, 128), lambda g, i: (0, g))
    bsp = pl.BlockSpec((1, 128), lambda g, i: (0, g))
    osp = pl.BlockSpec((RB, 128), lambda g, i: (g * nb + i, 0))
    return pl.pallas_call(
        _tc_b_body,
        grid=(2, nb),
        in_specs=[row, row, row, wsp, bsp, wsp, bsp, wsp, bsp, wsp, bsp],
        out_specs=[osp, osp, osp,
                   pl.BlockSpec((RB, 128), lambda g, i: (i, g))],
        out_shape=[_f32((2 * N, 128)), _f32((2 * N, 128)),
                   _f32((2 * N, 128)), _f32((N, C))],
    )(root, acc0, acc1, Wq, bq2, Wk, bk2, Wv, bv2, Ws, bs2)


@functools.partial(
    pl.kernel,
    out_type=[_f32((NPAD, 128)), _f32((NPAD, 128))],
    mesh=_mesh,
    compiler_params=_sc_params,
    scratch_types=[
        pltpu.VMEM_SHARED((NPAD, 128), jnp.float32),
        pltpu.VMEM_SHARED((NPAD,), jnp.float32),
        pltpu.VMEM_SHARED((NPAD,), jnp.float32),
        [pltpu.VMEM((CH,), jnp.int32)] * 2,
        [pltpu.VMEM((CH,), jnp.int32)] * 2,
        [pltpu.VMEM((CH,), jnp.int32)] * 2,
        [pltpu.VMEM((CH,), jnp.int32)] * 2,
        [pltpu.VMEM((CH, 128), jnp.float32)] * 2,
        [pltpu.VMEM((CH, 128), jnp.float32)] * 2,
        pltpu.VMEM((CH, 128), jnp.float32),
        pltpu.VMEM((CH, 128), jnp.float32),
        [pltpu.VMEM((CH,), jnp.float32)] * 2,
        [pltpu.VMEM((CH,), jnp.float32)] * 2,
        pltpu.VMEM((PZA, H1), jnp.float32),
        pltpu.VMEM((PZA,), jnp.float32),
        pltpu.VMEM((PZA,), jnp.float32),
        [pltpu.SemaphoreType.DMA] * 2,
        [pltpu.SemaphoreType.DMA] * 2,
        pltpu.SemaphoreType.DMA,
        pltpu.SemaphoreType.DMA,
    ],
)
def _sc_attn(src_h, dst_h, qf_h, kf_h, vf_h, za_h, out0, out1,
             att_sh, den0_sh, den1_sh, srcv, dstv, qidxv, sidxv,
             qr, kr, vr, scb, ebuf0, ebuf1, zav, d0v, d1v,
             isem, gsem, vsem, ssem):
    c = lax.axis_index("c")
    s = lax.axis_index("s")
    coff = c * N
    lane = lax.iota(jnp.int32, L)
    pltpu.sync_copy(za_h.at[pl.ds(0, PZA)], zav)
    for t in range(TROW // PZA):
        pltpu.sync_copy(zav, att_sh.at[pl.ds(s * TROW + t * PZA, PZA)])
    for j in range(PZA // L):
        d0v[pl.ds(j * L, L)] = jnp.zeros((L,), jnp.float32)
    for t in range(TROW // PZA):
        sl = pl.ds(s * TROW + t * PZA, PZA)
        pltpu.sync_copy(d0v, den0_sh.at[sl])
        pltpu.sync_copy(d0v, den1_sh.at[sl])
    plsc.subcore_barrier()

    NCH = EPT // CH
    base0 = s * EPT

    def load_idx(ch, u):
        b = base0 + ch * CH
        pltpu.async_copy(src_h.at[pl.ds(b, CH)], srcv[u], isem[u])
        pltpu.async_copy(dst_h.at[pl.ds(b, CH)], dstv[u], isem[u])

    def wait_idx(u):
        pltpu.make_async_copy(src_h.at[pl.ds(0, CH)], srcv[u],
                              isem[u]).wait()
        pltpu.make_async_copy(dst_h.at[pl.ds(0, CH)], dstv[u],
                              isem[u]).wait()

    def comp_idx(u):
        for j in range(CH // L):
            sl = pl.ds(j * L, L)
            qidxv[u][sl] = jnp.minimum(dstv[u][sl] + coff, 2 * N - 1)
            sidxv[u][sl] = jnp.minimum(srcv[u][sl] + coff, 2 * N - 1)

    def fire_qk(u):
        pltpu.async_copy(qf_h.at[qidxv[u]], qr[u], gsem[u])
        pltpu.async_copy(kf_h.at[sidxv[u]], kr[u], gsem[u])

    def wait_qk(u):
        pltpu.make_async_copy(qf_h.at[qidxv[u]], qr[u], gsem[u]).wait()
        pltpu.make_async_copy(kf_h.at[sidxv[u]], kr[u], gsem[u]).wait()

    def fire_v(u):
        pltpu.async_copy(vf_h.at[sidxv[u]], vr, vsem)

    def wait_v(u):
        pltpu.make_async_copy(vf_h.at[sidxv[u]], vr, vsem).wait()

    def fire_scat(u):
        pltpu.async_copy(scb, att_sh.at[dstv[u]], ssem, add=True)
        pltpu.async_copy(ebuf0[u], den0_sh.at[dstv[u]], ssem, add=True)
        pltpu.async_copy(ebuf1[u], den1_sh.at[dstv[u]], ssem, add=True)

    def wait_scat(u):
        pltpu.make_async_copy(scb, att_sh.at[dstv[u]], ssem).wait()
        pltpu.make_async_copy(ebuf0[u], den0_sh.at[dstv[u]], ssem).wait()
        pltpu.make_async_copy(ebuf1[u], den1_sh.at[dstv[u]], ssem).wait()

    def dots(u):
        def group(g, cc):
            eb0 = jnp.zeros((L,), jnp.float32)
            eb1 = jnp.zeros((L,), jnp.float32)
            for i in range(L):
                e = g * L + i
                qv = [qr[u][e, pl.ds(j * L, L)] for j in range(8)]
                kv = [kr[u][e, pl.ds(j * L, L)] for j in range(8)]
                p0 = (qv[0] * kv[0] + qv[1] * kv[1]
                      + qv[2] * kv[2] + qv[3] * kv[3])
                p1 = (qv[4] * kv[4] + qv[5] * kv[5]
                      + qv[6] * kv[6] + qv[7] * kv[7])
                e0 = jnp.exp(_hsum(p0, lane) * 0.125)
                e1 = jnp.exp(_hsum(p1, lane) * 0.125)
                eb0 = jnp.where(lane == i, e0, eb0)
                eb1 = jnp.where(lane == i, e1, eb1)
            ebuf0[u][pl.ds(g * L, L)] = eb0
            ebuf1[u][pl.ds(g * L, L)] = eb1
            return cc

        lax.fori_loop(0, CH // L, group, 0)

    def scale(u):
        def group(g, cc):
            e0g = ebuf0[u][pl.ds(g * L, L)]
            e1g = ebuf1[u][pl.ds(g * L, L)]
            for i in range(L):
                e = g * L + i
                e0 = _bcast(e0g, i, lane)
                e1 = _bcast(e1g, i, lane)
                for j in range(4):
                    sl = pl.ds(j * L, L)
                    scb[e, sl] = vr[e, sl] * e0
                for j in range(4, 8):
                    sl = pl.ds(j * L, L)
                    scb[e, sl] = vr[e, sl] * e1
            return cc

        lax.fori_loop(0, CH // L, group, 0)

    load_idx(0, 0)
    wait_idx(0)
    comp_idx(0)
    fire_qk(0)
    fire_v(0)
    load_idx(1, 1)

    def body(cpair, carry):
        i = 2 * cpair
        last = cpair == NCH // 2 - 1

        wait_idx(1)
        comp_idx(1)
        fire_qk(1)
        wait_qk(0)
        dots(0)

        @pl.when(cpair > 0)
        def _():
            wait_scat(1)
        wait_v(0)
        scale(0)
        fire_v(1)
        fire_scat(0)

        @pl.when(jnp.logical_not(last))
        def _():
            load_idx(i + 2, 0)

        @pl.when(jnp.logical_not(last))
        def _():
            wait_idx(0)
            comp_idx(0)
            fire_qk(0)
        wait_qk(1)
        dots(1)
        wait_scat(0)
        wait_v(1)
        scale(1)

        @pl.when(jnp.logical_not(last))
        def _():
            fire_v(0)
        fire_scat(1)

        @pl.when(jnp.logical_not(last))
        def _():
            load_idx(i + 3, 1)
        return carry

    lax.fori_loop(0, NCH // 2, body, 0)
    wait_scat(1)
    plsc.subcore_barrier()

    def piece(t, carry):
        rowbase = pl.multiple_of(s * TROW + t * PZA, PZA)
        sl = pl.ds(rowbase, PZA)
        pltpu.sync_copy(att_sh.at[sl], zav)
        pltpu.sync_copy(den0_sh.at[sl], d0v)
        pltpu.sync_copy(den1_sh.at[sl], d1v)

        def norm(g, cc):
            rows = g * L + lane
            i0 = 1.0 / jnp.maximum(d0v[pl.ds(g * L, L)], 1e-16)
            i1 = 1.0 / jnp.maximum(d1v[pl.ds(g * L, L)], 1e-16)
            for j in range(H2):
                cj = jnp.full((L,), j, jnp.int32)
                cj2 = jnp.full((L,), j + H2, jnp.int32)
                plsc.store_scatter(
                    zav, [rows, cj],
                    plsc.load_gather(zav, [rows, cj]) * i0)
                plsc.store_scatter(
                    zav, [rows, cj2],
                    plsc.load_gather(zav, [rows, cj2]) * i1)
            return cc

        lax.fori_loop(0, PZA // L, norm, 0)

        @pl.when(c == 0)
        def _():
            pltpu.sync_copy(zav, out0.at[sl])

        @pl.when(c == 1)
        def _():
            pltpu.sync_copy(zav, out1.at[sl])

        return carry

    lax.fori_loop(0, TROW // PZA, piece, 0)


def _tc_c_body(att0_ref, att1_ref, skip_ref, g_ref, b_ref, out_ref):
    out2 = jnp.concatenate([att0_ref[...], att1_ref[...]],
                           axis=1) + skip_ref[...]
    mu = jnp.mean(out2, axis=0, keepdims=True)
    var = jnp.mean(out2 * out2, axis=0, keepdims=True) - mu * mu
    xn = (out2 - mu) * lax.rsqrt(var + 1e-5)
    y = g_ref[...] * xn + b_ref[...]
    out_ref[...] = jnp.where(y > 0, y, 0.01 * y)


def _tc_c(att0, att1, skip, g2, b2):
    asp = pl.BlockSpec((N, 128), lambda i: (0, 0))
    return pl.pallas_call(
        _tc_c_body,
        grid=(1,),
        in_specs=[asp, asp,
                  pl.BlockSpec((N, C), lambda i: (0, 0)),
                  pl.BlockSpec((1, C), lambda i: (0, 0)),
                  pl.BlockSpec((1, C), lambda i: (0, 0))],
        out_specs=pl.BlockSpec((N, C), lambda i: (0, 0)),
        out_shape=_f32((N, C)),
    )(att0, att1, skip, g2, b2)


def kernel(node_features, edge_index, edge_type, W_rel, W_root, b_rgcn,
           Wq, bq, Wk, bk, Wv, bv, Wskip, bskip, gamma, beta):
    src = edge_index[0].astype(jnp.int32)
    dst = edge_index[1].astype(jnp.int32)
    rt = edge_type.astype(jnp.int32)
    pad = EP - E
    srcp = jnp.concatenate([src, jnp.zeros((pad,), jnp.int32)])
    dstp = jnp.concatenate([dst, jnp.full((pad,), N, jnp.int32)])
    rtp = jnp.concatenate([rt, jnp.zeros((pad,), jnp.int32)])

    zc = jnp.zeros((KSTRIPE,), jnp.float32)
    za = jnp.zeros((PZ, H1), jnp.float32)

    cnt0, cnt1 = _sc_count(dstp, rtp, zc)
    cnt2 = jnp.stack([cnt0, cnt1]).reshape(2, KPAD // 128, 128)

    xw, root = _tc_a1(node_features, W_rel, W_root, b_rgcn.reshape(1, H1))
    inv = _tc_a2(cnt2)
    xw2 = xw.reshape(R * N, H1)
    invf = inv.reshape(KPAD)

    acc0, acc1 = _sc_rgcn(srcp, dstp, rtp, xw2, invf, za)

    qf, kf, vf, skip = _tc_b(root, acc0, acc1,
                             Wq, bq.reshape(1, C), Wk, bk.reshape(1, C),
                             Wv, bv.reshape(1, C), Wskip, bskip.reshape(1, C))
    att0, att1 = _sc_attn(srcp, dstp, qf, kf, vf, za)

    return _tc_c(att0, att1, skip, gamma.reshape(1, C), beta.reshape(1, C))


# speedup vs baseline: 1.1180x; 1.0393x over previous
import functools

import jax
import jax.numpy as jnp
from jax import lax
from jax.experimental import pallas as pl
from jax.experimental.pallas import tpu as pltpu
from jax.experimental.pallas import tpu_sc as plsc

N = 10000
E = 320000
G = 128
H1 = 128
H2 = 64
HEADS = 4
R = 8
C = HEADS * H2

NC, NS, L = 2, 16, 16
NW = NC * NS

KSTRIPE = 5120
KPAD = NS * KSTRIPE
CH = 48
CHR = 80
EP = 322560
EPW = EP // NW
EPT = EP // NS
NPAD = 10240
TROW = NPAD // NS
PZ = 128
PZA = 16

_mesh = plsc.VectorSubcoreMesh(core_axis_name="c", subcore_axis_name="s")
_sc_params = pltpu.CompilerParams(needs_layout_passes=False)


def _f32(shape):
    return jax.ShapeDtypeStruct(shape, jnp.float32)


def _rot(v, k, lane):
    idx = jnp.bitwise_and(lane + k, L - 1)
    return v.at[idx].get(mode="promise_in_bounds")


def _bcast(v, i, lane):
    idx = jnp.bitwise_and(lane, 0) + i
    return v.at[idx].get(mode="promise_in_bounds")


def _hsum(v, lane):
    for k in (8, 4, 2, 1):
        v = v + _rot(v, k, lane)
    return v


@functools.partial(
    pl.kernel,
    out_type=[_f32((KPAD,)), _f32((KPAD,))],
    mesh=_mesh,
    compiler_params=_sc_params,
    scratch_types=[
        pltpu.VMEM_SHARED((KPAD,), jnp.float32),
        pltpu.VMEM((CH,), jnp.int32),
        pltpu.VMEM((CH,), jnp.int32),
        pltpu.VMEM((CH,), jnp.int32),
        pltpu.VMEM((CH,), jnp.float32),
        pltpu.VMEM((KSTRIPE,), jnp.float32),
        pltpu.SemaphoreType.DMA,
    ],
)
def _sc_count(dst_h, rt_h, zc_h, out0, out1, cnt_sh, dstv, rtv, keyv, onesv,
              zcv, sem):
    c = lax.axis_index("c")
    s = lax.axis_index("s")
    wid = c * NS + s
    for j in range(CH // L):
        onesv[pl.ds(j * L, L)] = jnp.full((L,), 1.0, jnp.float32)
    pltpu.sync_copy(zc_h, zcv)
    pltpu.sync_copy(zcv, cnt_sh.at[pl.ds(s * KSTRIPE, KSTRIPE)])
    plsc.subcore_barrier()

    def body(ch, carry):
        base = wid * EPW + ch * CH
        d1 = pltpu.async_copy(dst_h.at[pl.ds(base, CH)], dstv, sem)
        d2 = pltpu.async_copy(rt_h.at[pl.ds(base, CH)], rtv, sem)
        d1.wait()
        d2.wait()
        for j in range(CH // L):
            sl = pl.ds(j * L, L)
            keyv[sl] = dstv[sl] * R + rtv[sl]
        pltpu.sync_copy(onesv, cnt_sh.at[keyv], add=True)
        return carry

    lax.fori_loop(0, EPW // CH, body, 0)
    plsc.subcore_barrier()

    pltpu.sync_copy(cnt_sh.at[pl.ds(s * KSTRIPE, KSTRIPE)], zcv)

    @pl.when(c == 0)
    def _():
        pltpu.sync_copy(zcv, out0.at[pl.ds(s * KSTRIPE, KSTRIPE)])

    @pl.when(c == 1)
    def _():
        pltpu.sync_copy(zcv, out1.at[pl.ds(s * KSTRIPE, KSTRIPE)])


def _tc_a1_body(x_ref, wrel_ref, wroot_ref, b_ref, xw_ref, root_ref):
    r = pl.program_id(0)
    xw_ref[0] = jnp.dot(x_ref[...], wrel_ref[0],
                        preferred_element_type=jnp.float32)

    @pl.when(r == 0)
    def _():
        root_ref[...] = (
            jnp.dot(x_ref[...], wroot_ref[...],
                    preferred_element_type=jnp.float32) + b_ref[...])


def _tc_a1(x, W_rel, W_root, b2):
    return pl.pallas_call(
        _tc_a1_body,
        grid=(R,),
        in_specs=[
            pl.BlockSpec((N, G), lambda r: (0, 0)),
            pl.BlockSpec((1, G, H1), lambda r: (r, 0, 0)),
            pl.BlockSpec((G, H1), lambda r: (0, 0)),
            pl.BlockSpec((1, H1), lambda r: (0, 0)),
        ],
        out_specs=[
            pl.BlockSpec((1, N, H1), lambda r: (r, 0, 0)),
            pl.BlockSpec((N, H1), lambda r: (0, 0)),
        ],
        out_shape=[
            _f32((R, N, H1)),
            _f32((N, H1)),
        ],
    )(x, W_rel, W_root, b2)


def _tc_a2_body(cnt_ref, inv_ref):
    tot = cnt_ref[0] + cnt_ref[1]
    inv_ref[...] = 1.0 / jnp.maximum(tot, 1.0)


def _tc_a2(cnt2):
    return pl.pallas_call(
        _tc_a2_body,
        grid=(1,),
        in_specs=[pl.BlockSpec((2, KPAD // 128, 128), lambda i: (0, 0, 0))],
        out_specs=pl.BlockSpec((KPAD // 128, 128), lambda i: (0, 0)),
        out_shape=_f32((KPAD // 128, 128)),
    )(cnt2)


@functools.partial(
    pl.kernel,
    out_type=[_f32((NPAD, H1)), _f32((NPAD, H1))],
    mesh=_mesh,
    compiler_params=_sc_params,
    scratch_types=[
        pltpu.VMEM_SHARED((NPAD, H1), jnp.float32),
        [pltpu.VMEM((CHR,), jnp.int32)] * 2,
        [pltpu.VMEM((CHR,), jnp.int32)] * 2,
        [pltpu.VMEM((CHR,), jnp.int32)] * 2,
        [pltpu.VMEM((CHR,), jnp.int32)] * 2,
        [pltpu.VMEM((CHR,), jnp.int32)] * 2,
        [pltpu.VMEM((CHR,), jnp.float32)] * 2,
        [pltpu.VMEM((CHR, H1), jnp.float32)] * 2,
        [pltpu.VMEM((CHR, H1), jnp.float32)] * 2,
        pltpu.VMEM((PZA, H1), jnp.float32),
        [pltpu.SemaphoreType.DMA] * 2,
        [pltpu.SemaphoreType.DMA] * 2,
        pltpu.SemaphoreType.DMA,
    ],
)
def _sc_rgcn(src_h, dst_h, rt_h, xw_h, inv_h, za_h, out0, out1,
             acc_sh, srcv, dstv, rtv, gidxv, wkeyv, wv, rowsv, srow, zav,
             isem, gsem, ssem):
    c = lax.axis_index("c")
    s = lax.axis_index("s")
    wid = c * NS + s
    lane = lax.iota(jnp.int32, L)
    pltpu.sync_copy(za_h.at[pl.ds(0, PZA)], zav)
    for t in range(TROW // PZA):
        pltpu.sync_copy(zav, acc_sh.at[pl.ds(s * TROW + t * PZA, PZA)])
    plsc.subcore_barrier()

    NCH = EPW // CHR
    base0 = wid * EPW

    def load_idx(ch, u):
        b = base0 + ch * CHR
        pltpu.async_copy(src_h.at[pl.ds(b, CHR)], srcv[u], isem[u])
        pltpu.async_copy(dst_h.at[pl.ds(b, CHR)], dstv[u], isem[u])
        pltpu.async_copy(rt_h.at[pl.ds(b, CHR)], rtv[u], isem[u])

    def wait_idx(u):
        pltpu.make_async_copy(src_h.at[pl.ds(0, CHR)], srcv[u],
                              isem[u]).wait()
        pltpu.make_async_copy(dst_h.at[pl.ds(0, CHR)], dstv[u],
                              isem[u]).wait()
        pltpu.make_async_copy(rt_h.at[pl.ds(0, CHR)], rtv[u], isem[u]).wait()

    def comp_idx(u):
        for j in range(CHR // L):
            sl = pl.ds(j * L, L)
            gidxv[u][sl] = rtv[u][sl] * N + srcv[u][sl]
            wkeyv[u][sl] = dstv[u][sl] * R + rtv[u][sl]

    def fire_gath(u):
        pltpu.async_copy(xw_h.at[gidxv[u]], rowsv[u], gsem[u])
        pltpu.async_copy(inv_h.at[wkeyv[u]], wv[u], gsem[u])

    def wait_gath(u):
        pltpu.make_async_copy(xw_h.at[gidxv[u]], rowsv[u], gsem[u]).wait()
        pltpu.make_async_copy(inv_h.at[wkeyv[u]], wv[u], gsem[u]).wait()

    def fire_scat(u):
        pltpu.async_copy(srow[u], acc_sh.at[dstv[u]], ssem, add=True)

    def wait_scat(u):
        pltpu.make_async_copy(srow[u], acc_sh.at[dstv[u]], ssem).wait()

    def scale(u):
        def group(g, cc):
            wg = wv[u][pl.ds(g * L, L)]
            for i in range(L):
                e = g * L + i
                w = _bcast(wg, i, lane)
                for j in range(H1 // L):
                    sl = pl.ds(j * L, L)
                    srow[u][e, sl] = rowsv[u][e, sl] * w
            return cc

        lax.fori_loop(0, CHR // L, group, 0)

    load_idx(0, 0)
    wait_idx(0)
    comp_idx(0)
    fire_gath(0)
    load_idx(1, 1)

    def body(cpair, carry):
        i = 2 * cpair
        last = cpair == NCH // 2 - 1

        wait_idx(1)
        comp_idx(1)
        fire_gath(1)
        wait_gath(0)

        @pl.when(cpair > 0)
        def _():
            wait_scat(1)
        scale(0)
        fire_scat(0)

        @pl.when(jnp.logical_not(last))
        def _():
            load_idx(i + 2, 0)

        @pl.when(jnp.logical_not(last))
        def _():
            wait_idx(0)
            comp_idx(0)
            fire_gath(0)
        wait_gath(1)
        wait_scat(0)
        scale(1)
        fire_scat(1)

        @pl.when(jnp.logical_not(last))
        def _():
            load_idx(i + 3, 1)
        return carry

    lax.fori_loop(0, NCH // 2, body, 0)
    wait_scat(1)
    plsc.subcore_barrier()

    def piece(t, carry):
        rowbase = pl.multiple_of(s * TROW + t * PZA, PZA)
        sl = pl.ds(rowbase, PZA)
        pltpu.sync_copy(acc_sh.at[sl], zav)

        @pl.when(c == 0)
        def _():
            pltpu.sync_copy(zav, out0.at[sl])

        @pl.when(c == 1)
        def _():
            pltpu.sync_copy(zav, out1.at[sl])

        return carry

    lax.fori_loop(0, TROW // PZA, piece, 0)


def _tc_b_body(root_ref, a0_ref, a1_ref, wq_ref, bq_ref, wk_ref, bk_ref,
               wv_ref, bv_ref, ws_ref, bs_ref,
               q_ref, k_ref, v_ref, skip_ref):
    h = root_ref[...] + a0_ref[...] + a1_ref[...]
    q_ref[...] = jnp.dot(h, wq_ref[...],
                         preferred_element_type=jnp.float32) + bq_ref[...]
    k_ref[...] = jnp.dot(h, wk_ref[...],
                         preferred_element_type=jnp.float32) + bk_ref[...]
    v_ref[...] = jnp.dot(h, wv_ref[...],
                         preferred_element_type=jnp.float32) + bv_ref[...]
    skip_ref[...] = jnp.dot(h, ws_ref[...],
                            preferred_element_type=jnp.float32) + bs_ref[...]


def _tc_b(root, acc0, acc1, Wq, bq2, Wk, bk2, Wv, bv2, Ws, bs2):
    RB = 2000
    nb = N // RB
    row = pl.BlockSpec((RB, H1), lambda g, i: (i, 0))
    wsp = pl.BlockSpec((H1, 128), lambda g, i: (0, g))
    bsp = pl.BlockSpec((1, 128), lambda g, i: (0, g))
    osp = pl.BlockSpec((RB, 128), lambda g, i: (g * nb + i, 0))
    return pl.pallas_call(
        _tc_b_body,
        grid=(2, nb),
        in_specs=[row, row, row, wsp, bsp, wsp, bsp, wsp, bsp, wsp, bsp],
        out_specs=[osp, osp, osp,
                   pl.BlockSpec((RB, 128), lambda g, i: (i, g))],
        out_shape=[_f32((2 * N, 128)), _f32((2 * N, 128)),
                   _f32((2 * N, 128)), _f32((N, C))],
    )(root, acc0, acc1, Wq, bq2, Wk, bk2, Wv, bv2, Ws, bs2)


@functools.partial(
    pl.kernel,
    out_type=[_f32((NPAD, 128)), _f32((NPAD, 128))],
    mesh=_mesh,
    compiler_params=_sc_params,
    scratch_types=[
        pltpu.VMEM_SHARED((NPAD, 128), jnp.float32),
        pltpu.VMEM_SHARED((NPAD,), jnp.float32),
        pltpu.VMEM_SHARED((NPAD,), jnp.float32),
        [pltpu.VMEM((CH,), jnp.int32)] * 2,
        [pltpu.VMEM((CH,), jnp.int32)] * 2,
        [pltpu.VMEM((CH,), jnp.int32)] * 2,
        [pltpu.VMEM((CH,), jnp.int32)] * 2,
        [pltpu.VMEM((CH, 128), jnp.float32)] * 2,
        [pltpu.VMEM((CH, 128), jnp.float32)] * 2,
        pltpu.VMEM((CH, 128), jnp.float32),
        pltpu.VMEM((CH, 128), jnp.float32),
        [pltpu.VMEM((CH,), jnp.float32)] * 2,
        [pltpu.VMEM((CH,), jnp.float32)] * 2,
        pltpu.VMEM((PZA, H1), jnp.float32),
        pltpu.VMEM((PZA,), jnp.float32),
        pltpu.VMEM((PZA,), jnp.float32),
        [pltpu.SemaphoreType.DMA] * 2,
        [pltpu.SemaphoreType.DMA] * 2,
        pltpu.SemaphoreType.DMA,
        pltpu.SemaphoreType.DMA,
    ],
)
def _sc_attn(src_h, dst_h, qf_h, kf_h, vf_h, za_h, out0, out1,
             att_sh, den0_sh, den1_sh, srcv, dstv, qidxv, sidxv,
             qr, kr, vr, scb, ebuf0, ebuf1, zav, d0v, d1v,
             isem, gsem, vsem, ssem):
    c = lax.axis_index("c")
    s = lax.axis_index("s")
    coff = c * N
    lane = lax.iota(jnp.int32, L)
    pltpu.sync_copy(za_h.at[pl.ds(0, PZA)], zav)
    for t in range(TROW // PZA):
        pltpu.sync_copy(zav, att_sh.at[pl.ds(s * TROW + t * PZA, PZA)])
    for j in range(PZA // L):
        d0v[pl.ds(j * L, L)] = jnp.zeros((L,), jnp.float32)
    for t in range(TROW // PZA):
        sl = pl.ds(s * TROW + t * PZA, PZA)
        pltpu.sync_copy(d0v, den0_sh.at[sl])
        pltpu.sync_copy(d0v, den1_sh.at[sl])
    plsc.subcore_barrier()

    NCH = EPT // CH
    base0 = s * EPT

    def load_idx(ch, u):
        b = base0 + ch * CH
        pltpu.async_copy(src_h.at[pl.ds(b, CH)], srcv[u], isem[u])
        pltpu.async_copy(dst_h.at[pl.ds(b, CH)], dstv[u], isem[u])

    def wait_idx(u):
        pltpu.make_async_copy(src_h.at[pl.ds(0, CH)], srcv[u],
                              isem[u]).wait()
        pltpu.make_async_copy(dst_h.at[pl.ds(0, CH)], dstv[u],
                              isem[u]).wait()

    def comp_idx(u):
        for j in range(CH // L):
            sl = pl.ds(j * L, L)
            qidxv[u][sl] = jnp.minimum(dstv[u][sl] + coff, 2 * N - 1)
            sidxv[u][sl] = jnp.minimum(srcv[u][sl] + coff, 2 * N - 1)

    def fire_qk(u):
        pltpu.async_copy(qf_h.at[qidxv[u]], qr[u], gsem[u])
        pltpu.async_copy(kf_h.at[sidxv[u]], kr[u], gsem[u])

    def wait_qk(u):
        pltpu.make_async_copy(qf_h.at[qidxv[u]], qr[u], gsem[u]).wait()
        pltpu.make_async_copy(kf_h.at[sidxv[u]], kr[u], gsem[u]).wait()

    def fire_v(u):
        pltpu.async_copy(vf_h.at[sidxv[u]], vr, vsem)

    def wait_v(u):
        pltpu.make_async_copy(vf_h.at[sidxv[u]], vr, vsem).wait()

    def fire_scat(u):
        pltpu.async_copy(scb, att_sh.at[dstv[u]], ssem, add=True)
        pltpu.async_copy(ebuf0[u], den0_sh.at[dstv[u]], ssem, add=True)
        pltpu.async_copy(ebuf1[u], den1_sh.at[dstv[u]], ssem, add=True)

    def wait_scat(u):
        pltpu.make_async_copy(scb, att_sh.at[dstv[u]], ssem).wait()
        pltpu.make_async_copy(ebuf0[u], den0_sh.at[dstv[u]], ssem).wait()
        pltpu.make_async_copy(ebuf1[u], den1_sh.at[dstv[u]], ssem).wait()

    def dots(u):
        def group(g, cc):
            eb0 = jnp.zeros((L,), jnp.float32)
            eb1 = jnp.zeros((L,), jnp.float32)
            for i in range(L):
                e = g * L + i
                qv = [qr[u][e, pl.ds(j * L, L)] for j in range(8)]
                kv = [kr[u][e, pl.ds(j * L, L)] for j in range(8)]
                p0 = (qv[0] * kv[0] + qv[1] * kv[1]
                      + qv[2] * kv[2] + qv[3] * kv[3])
                p1 = (qv[4] * kv[4] + qv[5] * kv[5]
                      + qv[6] * kv[6] + qv[7] * kv[7])
                e0 = jnp.exp(_hsum(p0, lane) * 0.125)
                e1 = jnp.exp(_hsum(p1, lane) * 0.125)
                eb0 = jnp.where(lane == i, e0, eb0)
                eb1 = jnp.where(lane == i, e1, eb1)
            ebuf0[u][pl.ds(g * L, L)] = eb0
            ebuf1[u][pl.ds(g * L, L)] = eb1
            return cc

        lax.fori_loop(0, CH // L, group, 0)

    def scale(u):
        def group(g, cc):
            e0g = ebuf0[u][pl.ds(g * L, L)]
            e1g = ebuf1[u][pl.ds(g * L, L)]
            for i in range(L):
                e = g * L + i
                e0 = _bcast(e0g, i, lane)
                e1 = _bcast(e1g, i, lane)
                for j in range(4):
                    sl = pl.ds(j * L, L)
                    scb[e, sl] = vr[e, sl] * e0
                for j in range(4, 8):
                    sl = pl.ds(j * L, L)
                    scb[e, sl] = vr[e, sl] * e1
            return cc

        lax.fori_loop(0, CH // L, group, 0)

    load_idx(0, 0)
    wait_idx(0)
    comp_idx(0)
    fire_qk(0)
    fire_v(0)
    load_idx(1, 1)

    def body(cpair, carry):
        i = 2 * cpair
        last = cpair == NCH // 2 - 1

        wait_idx(1)
        comp_idx(1)
        fire_qk(1)
        wait_qk(0)
        dots(0)

        @pl.when(cpair > 0)
        def _():
            wait_scat(1)
        wait_v(0)
        scale(0)
        fire_v(1)
        fire_scat(0)

        @pl.when(jnp.logical_not(last))
        def _():
            load_idx(i + 2, 0)

        @pl.when(jnp.logical_not(last))
        def _():
            wait_idx(0)
            comp_idx(0)
            fire_qk(0)
        wait_qk(1)
        dots(1)
        wait_scat(0)
        wait_v(1)
        scale(1)

        @pl.when(jnp.logical_not(last))
        def _():
            fire_v(0)
        fire_scat(1)

        @pl.when(jnp.logical_not(last))
        def _():
            load_idx(i + 3, 1)
        return carry

    lax.fori_loop(0, NCH // 2, body, 0)
    wait_scat(1)
    plsc.subcore_barrier()

    def piece(t, carry):
        rowbase = pl.multiple_of(s * TROW + t * PZA, PZA)
        sl = pl.ds(rowbase, PZA)
        pltpu.sync_copy(att_sh.at[sl], zav)
        pltpu.sync_copy(den0_sh.at[sl], d0v)
        pltpu.sync_copy(den1_sh.at[sl], d1v)

        def norm(g, cc):
            rows = g * L + lane
            i0 = 1.0 / jnp.maximum(d0v[pl.ds(g * L, L)], 1e-16)
            i1 = 1.0 / jnp.maximum(d1v[pl.ds(g * L, L)], 1e-16)
            for j in range(H2):
                cj = jnp.full((L,), j, jnp.int32)
                cj2 = jnp.full((L,), j + H2, jnp.int32)
                plsc.store_scatter(
                    zav, [rows, cj],
                    plsc.load_gather(zav, [rows, cj]) * i0)
                plsc.store_scatter(
                    zav, [rows, cj2],
                    plsc.load_gather(zav, [rows, cj2]) * i1)
            return cc

        lax.fori_loop(0, PZA // L, norm, 0)

        @pl.when(c == 0)
        def _():
            pltpu.sync_copy(zav, out0.at[sl])

        @pl.when(c == 1)
        def _():
            pltpu.sync_copy(zav, out1.at[sl])

        return carry

    lax.fori_loop(0, TROW // PZA, piece, 0)


def _tc_c_body(att0_ref, att1_ref, skip_ref, g_ref, b_ref, out_ref):
    out2 = jnp.concatenate([att0_ref[...], att1_ref[...]],
                           axis=1) + skip_ref[...]
    mu = jnp.mean(out2, axis=0, keepdims=True)
    var = jnp.mean(out2 * out2, axis=0, keepdims=True) - mu * mu
    xn = (out2 - mu) * lax.rsqrt(var + 1e-5)
    y = g_ref[...] * xn + b_ref[...]
    out_ref[...] = jnp.where(y > 0, y, 0.01 * y)


def _tc_c(att0, att1, skip, g2, b2):
    asp = pl.BlockSpec((N, 128), lambda i: (0, 0))
    return pl.pallas_call(
        _tc_c_body,
        grid=(1,),
        in_specs=[asp, asp,
                  pl.BlockSpec((N, C), lambda i: (0, 0)),
                  pl.BlockSpec((1, C), lambda i: (0, 0)),
                  pl.BlockSpec((1, C), lambda i: (0, 0))],
        out_specs=pl.BlockSpec((N, C), lambda i: (0, 0)),
        out_shape=_f32((N, C)),
    )(att0, att1, skip, g2, b2)


def kernel(node_features, edge_index, edge_type, W_rel, W_root, b_rgcn,
           Wq, bq, Wk, bk, Wv, bv, Wskip, bskip, gamma, beta):
    src = edge_index[0].astype(jnp.int32)
    dst = edge_index[1].astype(jnp.int32)
    rt = edge_type.astype(jnp.int32)
    pad = EP - E
    srcp = jnp.concatenate([src, jnp.zeros((pad,), jnp.int32)])
    dstp = jnp.concatenate([dst, jnp.full((pad,), N, jnp.int32)])
    rtp = jnp.concatenate([rt, jnp.zeros((pad,), jnp.int32)])

    zc = jnp.zeros((KSTRIPE,), jnp.float32)
    za = jnp.zeros((PZ, H1), jnp.float32)

    cnt0, cnt1 = _sc_count(dstp, rtp, zc)
    cnt2 = jnp.stack([cnt0, cnt1]).reshape(2, KPAD // 128, 128)

    xw, root = _tc_a1(node_features, W_rel, W_root, b_rgcn.reshape(1, H1))
    inv = _tc_a2(cnt2)
    xw2 = xw.reshape(R * N, H1)
    invf = inv.reshape(KPAD)

    acc0, acc1 = _sc_rgcn(srcp, dstp, rtp, xw2, invf, za)

    qf, kf, vf, skip = _tc_b(root, acc0, acc1,
                             Wq, bq.reshape(1, C), Wk, bk.reshape(1, C),
                             Wv, bv.reshape(1, C), Wskip, bskip.reshape(1, C))
    att0, att1 = _sc_attn(srcp, dstp, qf, kf, vf, za)

    return _tc_c(att0, att1, skip, gamma.reshape(1, C), beta.reshape(1, C))
